# baseline (speedup 1.0000x reference)
import jax
import jax.numpy as jnp
from jax.experimental import pallas as pl
from jax.experimental.pallas import tpu as pltpu
from jax.experimental.pallas import tpu_sc as plsc

_RB = 4
_L = 16


def kernel(x, pe_table):
    B, S, H = x.shape

    mesh = plsc.VectorSubcoreMesh(core_axis_name="c", subcore_axis_name="s")

    @pl.kernel(out_type=jax.ShapeDtypeStruct((B, S, H), x.dtype), mesh=mesh)
    def pe_add_sc(x_hbm, pe_hbm, o_hbm):
        def body(x_vmem, pe_vmem, o_vmem):
            for r in range(_RB):

                @plsc.parallel_loop(0, H, step=_L, unroll=8)
                def _chunk(col, _r=r):
                    slc = pl.ds(col, _L)
                    pe_chunk = pe_vmem.at[_r].at[slc][...]
                    for b in range(B):
                        o_vmem.at[b].at[_r].at[slc][...] = (
                            x_vmem.at[b].at[_r].at[slc][...] + pe_chunk
                        )

        n_workers = 32
        steps = S // _RB // n_workers
        pltpu.emit_pipeline(
            body,
            grid=(n_workers, steps),
            in_specs=[
                pl.BlockSpec(
                    (B, _RB, H), lambda w, j: (0, w * steps + j, 0)
                ),
                pl.BlockSpec((_RB, H), lambda w, j: (w * steps + j, 0)),
            ],
            out_specs=[
                pl.BlockSpec(
                    (B, _RB, H), lambda w, j: (0, w * steps + j, 0)
                )
            ],
            core_axis_name=("c", "s"),
            dimension_semantics=(pltpu.PARALLEL, pltpu.ARBITRARY),
            trace_scopes=False,
        )(x_hbm, pe_hbm, o_hbm)

    return pe_add_sc(x, pe_table)


# speedup vs baseline: 1.0189x; 1.0189x over previous
import jax
import jax.numpy as jnp
from jax import lax
from jax.experimental import pallas as pl
from jax.experimental.pallas import tpu as pltpu
from jax.experimental.pallas import tpu_sc as plsc

_L = 16
_NW = 32
_RB = 2
_NBUF = 4


def kernel(x, pe_table):
    B, S, H = x.shape
    rows_per_w = S // _NW
    steps = rows_per_w // _RB
    groups = steps // _NBUF

    mesh = plsc.VectorSubcoreMesh(core_axis_name="c", subcore_axis_name="s")

    @pl.kernel(
        out_type=jax.ShapeDtypeStruct((B, S, H), x.dtype),
        mesh=mesh,
        scratch_types=[
            pltpu.VMEM((_NBUF, B, _RB, H), jnp.float32),
            pltpu.VMEM((_NBUF, _RB, H), jnp.float32),
            pltpu.VMEM((_NBUF, B, _RB, H), jnp.float32),
            pltpu.SemaphoreType.DMA((_NBUF,)),
            pltpu.SemaphoreType.DMA((_NBUF,)),
        ],
    )
    def pe_add_sc(x_hbm, pe_hbm, o_hbm, xb, peb, ob, insem, outsem):
        wid = lax.axis_index("c") * 16 + lax.axis_index("s")
        base = wid * rows_per_w

        def in_copies(row, k):
            cx = pltpu.make_async_copy(
                x_hbm.at[:, pl.ds(row, _RB), :], xb.at[k], insem.at[k]
            )
            cpe = pltpu.make_async_copy(
                pe_hbm.at[pl.ds(row, _RB), :], peb.at[k], insem.at[k]
            )
            return cx, cpe

        for k in range(_NBUF):
            cx, cpe = in_copies(base + k * _RB, k)
            cx.start()
            cpe.start()

        @pl.loop(0, groups)
        def _group(g):
            for k in range(_NBUF):
                row = base + (g * _NBUF + k) * _RB
                cx, cpe = in_copies(row, k)
                cx.wait()
                cpe.wait()

                cout = pltpu.make_async_copy(
                    ob.at[k], o_hbm.at[:, pl.ds(row, _RB), :], outsem.at[k]
                )

                @pl.when(g > 0)
                def _drain():
                    pltpu.make_async_copy(
                        ob.at[k],
                        o_hbm.at[:, pl.ds(row, _RB), :],
                        outsem.at[k],
                    ).wait()

                for r in range(_RB):

                    @plsc.parallel_loop(0, H, step=_L, unroll=4)
                    def _chunk(col, _r=r, _k=k):
                        slc = pl.ds(col, _L)
                        pe_chunk = peb.at[_k].at[_r].at[slc][...]
                        for b in range(B):
                            ob.at[_k].at[b].at[_r].at[slc][...] = (
                                xb.at[_k].at[b].at[_r].at[slc][...] + pe_chunk
                            )

                cout.start()

                @pl.when(g < groups - 1)
                def _refill():
                    nrow = base + ((g + 1) * _NBUF + k) * _RB
                    ncx, ncpe = in_copies(nrow, k)
                    ncx.start()
                    ncpe.start()

        for k in range(_NBUF):
            row = base + ((groups - 1) * _NBUF + k) * _RB
            pltpu.make_async_copy(
                ob.at[k], o_hbm.at[:, pl.ds(row, _RB), :], outsem.at[k]
            ).wait()

    return pe_add_sc(x, pe_table)
